# baseline (speedup 1.0000x reference)
import jax
import jax.numpy as jnp
from jax import lax
from jax.experimental import pallas as pl
from jax.experimental.pallas import tpu as pltpu

B, S, D_MODEL, D_FEAT = 4, 4096, 2048, 256
TARGET_SNR = 0.3
LN_EPS = 1e-5
BS = 1024
NB = B * S // BS


def _body(last_ref, x_ref, ff_ref, g_ref, beta_ref, w_hbm, bias_ref, out_ref,
          w_vmem, w_sem):
    i = pl.program_id(0)
    b = i // (S // BS)
    r = b * S + last_ref[b]
    jb = r // BS
    off = lax.rem(r, BS)

    out_ref[...] = x_ref[...]

    @pl.when(i == jb)
    def _():
        cp = pltpu.make_async_copy(w_hbm, w_vmem, w_sem)
        cp.start()
        ff = ff_ref[0]
        mean = jnp.mean(ff, axis=-1, keepdims=True)
        cent = ff - mean
        var = jnp.mean(cent * cent, axis=-1, keepdims=True)
        ln = cent * lax.rsqrt(var + LN_EPS) * g_ref[...] + beta_ref[...]
        cp.wait()
        proj = lax.dot_general(ln, w_vmem[...], (((1,), (1,)), ((), ())),
                               preferred_element_type=jnp.float32)
        proj = proj + bias_ref[...]
        nrm = jnp.sqrt(jnp.sum(proj * proj, axis=-1, keepdims=True))
        direction = proj / jnp.maximum(nrm, 1e-12)
        host = x_ref[pl.ds(off, 1), :]
        rms = jnp.sqrt(jnp.mean(host * host, axis=-1, keepdims=True))
        out_ref[pl.ds(off, 1), :] = host + direction * (rms * TARGET_SNR)


def kernel(x, faculty_features, ln_gamma, ln_beta, W, b, token_ids,
           last_indices):
    del token_ids
    last = last_indices.astype(jnp.int32)

    grid_spec = pltpu.PrefetchScalarGridSpec(
        num_scalar_prefetch=1,
        grid=(NB,),
        in_specs=[
            pl.BlockSpec((BS, D_MODEL), lambda i, last_ref: (i, 0)),
            pl.BlockSpec((1, 1, D_FEAT),
                         lambda i, last_ref: (i // (S // BS), 0, 0)),
            pl.BlockSpec((1, D_FEAT), lambda i, last_ref: (0, 0)),
            pl.BlockSpec((1, D_FEAT), lambda i, last_ref: (0, 0)),
            pl.BlockSpec(memory_space=pltpu.MemorySpace.HBM),
            pl.BlockSpec((1, D_MODEL), lambda i, last_ref: (0, 0)),
        ],
        out_specs=pl.BlockSpec((BS, D_MODEL), lambda i, last_ref: (i, 0)),
        scratch_shapes=[
            pltpu.VMEM((D_MODEL, D_FEAT), jnp.float32),
            pltpu.SemaphoreType.DMA,
        ],
    )

    out2d = pl.pallas_call(
        _body,
        grid_spec=grid_spec,
        out_shape=jax.ShapeDtypeStruct((B * S, D_MODEL), jnp.float32),
        compiler_params=pltpu.CompilerParams(vmem_limit_bytes=134217728),
    )(last, x.reshape(B * S, D_MODEL), faculty_features.reshape(B, 1, D_FEAT),
      ln_gamma.reshape(1, D_FEAT), ln_beta.reshape(1, D_FEAT), W,
      b.reshape(1, D_MODEL))
    return out2d.reshape(B, S, D_MODEL)


# speedup vs baseline: 1.0948x; 1.0948x over previous
import jax
import jax.numpy as jnp
from jax import lax
from jax.experimental import pallas as pl
from jax.experimental.pallas import tpu as pltpu

B, S, D_MODEL, D_FEAT = 4, 4096, 2048, 256
TARGET_SNR = 0.3
LN_EPS = 1e-5
BS = 1024
NB = B * S // BS


def _body(last_ref, x_ref, ff_ref, g_ref, beta_ref, w_ref, bias_ref, out_ref):
    i = pl.program_id(0)
    b = i // (S // BS)
    r = b * S + last_ref[b]
    jb = r // BS
    off = lax.rem(r, BS)

    out_ref[...] = x_ref[...]

    @pl.when(i == jb)
    def _():
        ff = ff_ref[0]
        mean = jnp.mean(ff, axis=-1, keepdims=True)
        cent = ff - mean
        var = jnp.mean(cent * cent, axis=-1, keepdims=True)
        ln = cent * lax.rsqrt(var + LN_EPS) * g_ref[...] + beta_ref[...]
        proj = lax.dot_general(ln, w_ref[...], (((1,), (1,)), ((), ())),
                               preferred_element_type=jnp.float32)
        proj = proj + bias_ref[...]
        nrm = jnp.sqrt(jnp.sum(proj * proj, axis=-1, keepdims=True))
        direction = proj / jnp.maximum(nrm, 1e-12)
        host = x_ref[pl.ds(off, 1), :]
        rms = jnp.sqrt(jnp.mean(host * host, axis=-1, keepdims=True))
        out_ref[pl.ds(off, 1), :] = host + direction * (rms * TARGET_SNR)


def kernel(x, faculty_features, ln_gamma, ln_beta, W, b, token_ids,
           last_indices):
    del token_ids
    last = last_indices.astype(jnp.int32)

    grid_spec = pltpu.PrefetchScalarGridSpec(
        num_scalar_prefetch=1,
        grid=(NB,),
        in_specs=[
            pl.BlockSpec((BS, D_MODEL), lambda i, last_ref: (i, 0)),
            pl.BlockSpec((1, 1, D_FEAT),
                         lambda i, last_ref: (i // (S // BS), 0, 0)),
            pl.BlockSpec((1, D_FEAT), lambda i, last_ref: (0, 0)),
            pl.BlockSpec((1, D_FEAT), lambda i, last_ref: (0, 0)),
            pl.BlockSpec((D_MODEL, D_FEAT), lambda i, last_ref: (0, 0)),
            pl.BlockSpec((1, D_MODEL), lambda i, last_ref: (0, 0)),
        ],
        out_specs=pl.BlockSpec((BS, D_MODEL), lambda i, last_ref: (i, 0)),
    )

    out2d = pl.pallas_call(
        _body,
        grid_spec=grid_spec,
        out_shape=jax.ShapeDtypeStruct((B * S, D_MODEL), jnp.float32),
    )(last, x.reshape(B * S, D_MODEL), faculty_features.reshape(B, 1, D_FEAT),
      ln_gamma.reshape(1, D_FEAT), ln_beta.reshape(1, D_FEAT), W,
      b.reshape(1, D_MODEL))
    return out2d.reshape(B, S, D_MODEL)
